# baseline (speedup 1.0000x reference)
import functools

import jax
import jax.numpy as jnp
from jax import lax
from jax.experimental import pallas as pl
from jax.experimental.pallas import tpu as pltpu
from jax.experimental.pallas import tpu_sc as plsc

E = 23
K = 3
D = 768
H = 768
N = 8192
EL = 128
BLK = 256
G = 118
NSLOT = G * BLK
TN = 1024
PB = (N * K) // BLK

NEG = -1e30

_INV_SQRT2 = 0.7071067811865476


def _gelu_exact(v):
    return 0.5 * v * (1.0 + lax.erf(v * _INV_SQRT2))


def _gating_body(x_ref, gw_ref, gb_ref, bias_ref, ti_ref, tw_ref, cnt_ref,
                 ps_ref):
    step = pl.program_id(0)
    x = x_ref[...]
    logits = jnp.dot(x, gw_ref[...], preferred_element_type=jnp.float32)
    logits = logits + gb_ref[...]
    lane = lax.broadcasted_iota(jnp.int32, (TN, EL), 1)
    valid = lane < E
    g = jnp.where(valid, jax.nn.sigmoid(logits), 0.0)
    cur = jnp.where(valid, g + bias_ref[...], NEG)
    ams = []
    wks = []
    for _ in range(K):
        m = jnp.max(cur, axis=1, keepdims=True)
        am = jnp.min(jnp.where(cur == m, lane, EL), axis=1, keepdims=True)
        sel = lane == am
        wk = jnp.sum(jnp.where(sel, g, 0.0), axis=1, keepdims=True)
        ams.append(am)
        wks.append(wk)
        cur = jnp.where(sel, NEG, cur)
    wsum = wks[0] + wks[1] + wks[2]
    col8 = lax.broadcasted_iota(jnp.int32, (TN, 8), 1)
    ti = jnp.where(col8 == 0, ams[0],
                   jnp.where(col8 == 1, ams[1],
                             jnp.where(col8 == 2, ams[2], 0)))
    tw = jnp.where(col8 == 0, wks[0] / wsum,
                   jnp.where(col8 == 1, wks[1] / wsum,
                             jnp.where(col8 == 2, wks[2] / wsum, 0.0)))
    ti_ref[...] = ti.astype(jnp.int32)
    tw_ref[...] = tw
    cnts = jnp.zeros((1, EL), jnp.int32)
    for k in range(K):
        oh = (lane == ams[k]).astype(jnp.int32)
        cnts = cnts + jnp.sum(oh, axis=0, keepdims=True)
    cntb = jnp.broadcast_to(cnts, (8, EL)).astype(jnp.int32)
    gn = g / jnp.sum(g, axis=1, keepdims=True)
    psb = jnp.broadcast_to(jnp.sum(gn, axis=0, keepdims=True), (8, EL))

    @pl.when(step == 0)
    def _():
        cnt_ref[...] = cntb
        ps_ref[...] = psb

    @pl.when(step != 0)
    def _():
        cnt_ref[...] = cnt_ref[...] + cntb
        ps_ref[...] = ps_ref[...] + psb


def _gating(xf, gate_w_p, gate_b_p, bias_p):
    nb = N // TN
    return pl.pallas_call(
        _gating_body,
        grid=(nb,),
        in_specs=[
            pl.BlockSpec((TN, D), lambda i: (i, 0)),
            pl.BlockSpec((D, EL), lambda i: (0, 0)),
            pl.BlockSpec((1, EL), lambda i: (0, 0)),
            pl.BlockSpec((1, EL), lambda i: (0, 0)),
        ],
        out_specs=[
            pl.BlockSpec((TN, 8), lambda i: (i, 0)),
            pl.BlockSpec((TN, 8), lambda i: (i, 0)),
            pl.BlockSpec((8, EL), lambda i: (0, 0)),
            pl.BlockSpec((8, EL), lambda i: (0, 0)),
        ],
        out_shape=[
            jax.ShapeDtypeStruct((N, 8), jnp.int32),
            jax.ShapeDtypeStruct((N, 8), jnp.float32),
            jax.ShapeDtypeStruct((8, EL), jnp.int32),
            jax.ShapeDtypeStruct((8, EL), jnp.float32),
        ],
    )(xf, gate_w_p, gate_b_p, bias_p)


def _routing_body(ti_ref, cnt_ref, ps_ref, slots_ref, be_ref, aux_ref,
                  carry_ref, offs_ref):
    step = pl.program_id(0)

    @pl.when(step == 0)
    def _():
        cnt_row = cnt_ref[0:1, :].astype(jnp.float32)
        padded = jnp.ceil(cnt_row / BLK) * BLK
        r0 = lax.broadcasted_iota(jnp.int32, (EL, EL), 0)
        r1 = lax.broadcasted_iota(jnp.int32, (EL, EL), 1)
        eye = (r0 == r1).astype(jnp.float32)
        padded_col = lax.dot_general(
            eye, padded, (((1,), (1,)), ((), ())),
            preferred_element_type=jnp.float32)
        pc32 = padded_col[:32, :]
        s0 = lax.broadcasted_iota(jnp.int32, (32, 32), 0)
        s1 = lax.broadcasted_iota(jnp.int32, (32, 32), 1)
        tri = (s1 < s0).astype(jnp.float32)
        offs_col = jnp.dot(tri, pc32,
                           preferred_element_type=jnp.float32)
        offs_ref[...] = jnp.broadcast_to(offs_col, (32, EL))
        carry_ref[...] = jnp.zeros((32, EL), jnp.float32)
        offs_e = jnp.broadcast_to(offs_col[:E + 9, :], (32, EL))
        gmul = lax.broadcasted_iota(jnp.int32, (32, EL), 1).astype(
            jnp.float32) * BLK
        sub = lax.broadcasted_iota(jnp.int32, (32, EL), 0)
        cmp = jnp.where((sub < E) & (offs_ref[...] <= gmul), 1.0, 0.0)
        be_row = jnp.sum(cmp, axis=0, keepdims=True) - 1.0
        be_row = jnp.clip(be_row, 0.0, float(E - 1))
        be_ref[...] = jnp.broadcast_to(be_row, (8, EL)).astype(jnp.int32)
        lane = lax.broadcasted_iota(jnp.int32, (1, EL), 1)
        p_mean = ps_ref[0:1, :] / N
        fv = (E * cnt_row) / (K * N)
        aux = jnp.sum(jnp.where(lane < E, p_mean * fv, 0.0))
        aux_ref[...] = jnp.full((8, EL), aux, jnp.float32)

    e_row = ti_ref[...].reshape(1, BLK)
    sub_e = lax.broadcasted_iota(jnp.int32, (32, BLK), 0)
    oh = (sub_e == e_row).astype(jnp.float32)
    p0 = lax.broadcasted_iota(jnp.int32, (BLK, BLK), 0)
    p1 = lax.broadcasted_iota(jnp.int32, (BLK, BLK), 1)
    tri = (p0 < p1).astype(jnp.float32)
    prefix = jnp.dot(oh, tri, preferred_element_type=jnp.float32)
    rank_in = jnp.sum(oh * prefix, axis=0, keepdims=True)
    base = jnp.sum(
        oh * (carry_ref[:, 0:1] + offs_ref[:, 0:1]), axis=0, keepdims=True)
    slot = (rank_in + base).astype(jnp.int32)
    slots_ref[...] = slot.reshape(1, 1, BLK)
    carry_new = carry_ref[:, 0:1] + jnp.sum(oh, axis=1, keepdims=True)
    carry_ref[...] = jnp.broadcast_to(carry_new, (32, EL))


def _routing(ti_k, cnt8, ps8):
    return pl.pallas_call(
        _routing_body,
        grid=(PB,),
        in_specs=[
            pl.BlockSpec((1, 1, BLK), lambda s: (s, 0, 0)),
            pl.BlockSpec((8, EL), lambda s: (0, 0)),
            pl.BlockSpec((8, EL), lambda s: (0, 0)),
        ],
        out_specs=[
            pl.BlockSpec((1, 1, BLK), lambda s: (s, 0, 0)),
            pl.BlockSpec((8, EL), lambda s: (0, 0)),
            pl.BlockSpec((8, EL), lambda s: (0, 0)),
        ],
        out_shape=[
            jax.ShapeDtypeStruct((PB, 1, BLK), jnp.int32),
            jax.ShapeDtypeStruct((8, EL), jnp.int32),
            jax.ShapeDtypeStruct((8, EL), jnp.float32),
        ],
        scratch_shapes=[
            pltpu.VMEM((32, EL), jnp.float32),
            pltpu.VMEM((32, EL), jnp.float32),
        ],
    )(ti_k, cnt8, ps8)


def _expert_body(be_ref, x_ref, w1_ref, b1_ref, w2_ref, b2_ref, y_ref):
    x = x_ref[...]
    h = jnp.dot(x, w1_ref[0], preferred_element_type=jnp.float32) + b1_ref[0]
    h = _gelu_exact(h)
    y_ref[...] = (jnp.dot(h, w2_ref[0], preferred_element_type=jnp.float32)
                  + b2_ref[0])


def _expert_mlp(be, padded_x, uw1, ub1r, uw2, ub2r):
    grid_spec = pltpu.PrefetchScalarGridSpec(
        num_scalar_prefetch=1,
        grid=(G,),
        in_specs=[
            pl.BlockSpec((BLK, D), lambda g, be: (g, 0)),
            pl.BlockSpec((1, D, H), lambda g, be: (be[g], 0, 0)),
            pl.BlockSpec((1, 1, H), lambda g, be: (be[g], 0, 0)),
            pl.BlockSpec((1, H, D), lambda g, be: (be[g], 0, 0)),
            pl.BlockSpec((1, 1, D), lambda g, be: (be[g], 0, 0)),
        ],
        out_specs=pl.BlockSpec((BLK, D), lambda g, be: (g, 0)),
    )
    return pl.pallas_call(
        _expert_body,
        grid_spec=grid_spec,
        out_shape=jax.ShapeDtypeStruct((NSLOT, D), jnp.float32),
    )(be, padded_x, uw1, ub1r, uw2, ub2r)


_SB = 512


def _shared_body(x_ref, w1_ref, b1_ref, w2_ref, b2_ref, s_ref):
    x = x_ref[...]
    h = jnp.dot(x, w1_ref[...], preferred_element_type=jnp.float32) + b1_ref[...]
    h = _gelu_exact(h)
    s_ref[...] = (jnp.dot(h, w2_ref[...], preferred_element_type=jnp.float32)
                  + b2_ref[...])


def _shared(xf, sw1, sb1r, sw2, sb2r):
    return pl.pallas_call(
        _shared_body,
        grid=(N // _SB,),
        in_specs=[
            pl.BlockSpec((_SB, D), lambda i: (i, 0)),
            pl.BlockSpec((D, H), lambda i: (0, 0)),
            pl.BlockSpec((1, H), lambda i: (0, 0)),
            pl.BlockSpec((H, D), lambda i: (0, 0)),
            pl.BlockSpec((1, D), lambda i: (0, 0)),
        ],
        out_specs=pl.BlockSpec((_SB, D), lambda i: (i, 0)),
        out_shape=jax.ShapeDtypeStruct((N, D), jnp.float32),
    )(xf, sw1, sb1r, sw2, sb2r)


def _wadd_body(s_ref, yg_ref, tw_ref, o_ref):
    tw = tw_ref[...]
    o_ref[...] = (s_ref[...]
                  + tw[:, 0:1] * yg_ref[0]
                  + tw[:, 1:2] * yg_ref[1]
                  + tw[:, 2:3] * yg_ref[2])


def _wadd(s, yg, tw8):
    return pl.pallas_call(
        _wadd_body,
        grid=(N // _SB,),
        in_specs=[
            pl.BlockSpec((_SB, D), lambda i: (i, 0)),
            pl.BlockSpec((K, _SB, D), lambda i: (0, i, 0)),
            pl.BlockSpec((_SB, 8), lambda i: (i, 0)),
        ],
        out_specs=pl.BlockSpec((_SB, D), lambda i: (i, 0)),
        out_shape=jax.ShapeDtypeStruct((N, D), jnp.float32),
    )(s, yg, tw8)


_NC = 2
_NS = 16
_NW = _NC * _NS
TPW = N // _NW
CT = 64


def _dispatch_body(x_hbm, slots_hbm, out_hbm, rows_v, idx_v, sem):
    wid = lax.axis_index("s") * _NC + lax.axis_index("c")
    base = wid * TPW
    for c in range(TPW // CT):
        b = base + c * CT
        pltpu.sync_copy(x_hbm.at[pl.ds(b, CT), :], rows_v)
        for k in range(K):
            pltpu.sync_copy(slots_hbm.at[k, pl.ds(b, CT)], idx_v)
            pltpu.async_copy(rows_v, out_hbm.at[idx_v], sem).wait()


_dispatch = functools.partial(
    pl.kernel,
    mesh=plsc.VectorSubcoreMesh(core_axis_name="c", subcore_axis_name="s"),
    out_type=jax.ShapeDtypeStruct((NSLOT, D), jnp.float32),
    scratch_types=[
        pltpu.VMEM((CT, D), jnp.float32),
        pltpu.VMEM((CT,), jnp.int32),
        pltpu.SemaphoreType.DMA,
    ],
)(_dispatch_body)


def _combine_body(y_hbm, slots_hbm, out_hbm, idx_v, rows_v, sem):
    wid = lax.axis_index("s") * _NC + lax.axis_index("c")
    base = wid * TPW
    for c in range(TPW // CT):
        b = base + c * CT
        for k in range(K):
            pltpu.sync_copy(slots_hbm.at[k, pl.ds(b, CT)], idx_v)
            pltpu.async_copy(y_hbm.at[idx_v], rows_v, sem).wait()
            pltpu.sync_copy(rows_v, out_hbm.at[k, pl.ds(b, CT), :])


_combine = functools.partial(
    pl.kernel,
    mesh=plsc.VectorSubcoreMesh(core_axis_name="c", subcore_axis_name="s"),
    out_type=jax.ShapeDtypeStruct((K, N, D), jnp.float32),
    scratch_types=[
        pltpu.VMEM((CT,), jnp.int32),
        pltpu.VMEM((CT, D), jnp.float32),
        pltpu.SemaphoreType.DMA,
    ],
)(_combine_body)


def kernel(x, gate_w, gate_b, bias_buf, uw1, ub1, uw2, ub2, sw1, sb1, sw2,
           sb2):
    o_shape = x.shape
    xf = x.reshape(N, D)

    gate_w_p = jnp.zeros((D, EL), jnp.float32).at[:, :E].set(gate_w)
    gate_b_p = jnp.zeros((1, EL), jnp.float32).at[:, :E].set(gate_b)
    bias_p = jnp.full((1, EL), NEG, jnp.float32).at[:, :E].set(bias_buf)

    s = _shared(xf, sw1, sb1.reshape(1, H), sw2, sb2.reshape(1, D))

    ti8, tw8, cnt8, ps8 = _gating(xf, gate_w_p, gate_b_p, bias_p)

    ti_k = jnp.transpose(ti8[:, :K]).reshape(PB, 1, BLK)
    slots_b, be8, aux8 = _routing(ti_k, cnt8, ps8)
    slots = slots_b.reshape(K, N)
    be = be8[0, :G]

    padded_x = _dispatch(xf, slots)
    y = _expert_mlp(be, padded_x, uw1, ub1.reshape(E, 1, H), uw2,
                    ub2.reshape(E, 1, D))
    yg = _combine(y, slots)
    out = _wadd(s, yg, tw8)

    return (out.reshape(o_shape), aux8[0, 0], cnt8[0, :E])


# speedup vs baseline: 1.0516x; 1.0516x over previous
import functools

import jax
import jax.numpy as jnp
from jax import lax
from jax.experimental import pallas as pl
from jax.experimental.pallas import tpu as pltpu
from jax.experimental.pallas import tpu_sc as plsc

E = 23
K = 3
D = 768
H = 768
N = 8192
EL = 128
BLK = 256
G = 118
NSLOT = G * BLK
TN = 1024
PB = (N * K) // BLK

NEG = -1e30

_INV_SQRT2 = 0.7071067811865476


def _gelu_exact(v):
    return 0.5 * v * (1.0 + lax.erf(v * _INV_SQRT2))


def _gating_body(x_ref, gw_ref, gb_ref, bias_ref, ti_ref, tw_ref, cnt_ref,
                 ps_ref):
    step = pl.program_id(0)
    x = x_ref[...]
    logits = jnp.dot(x, gw_ref[...], preferred_element_type=jnp.float32)
    logits = logits + gb_ref[...]
    lane = lax.broadcasted_iota(jnp.int32, (TN, EL), 1)
    valid = lane < E
    g = jnp.where(valid, jax.nn.sigmoid(logits), 0.0)
    cur = jnp.where(valid, g + bias_ref[...], NEG)
    ams = []
    wks = []
    for _ in range(K):
        m = jnp.max(cur, axis=1, keepdims=True)
        am = jnp.min(jnp.where(cur == m, lane, EL), axis=1, keepdims=True)
        sel = lane == am
        wk = jnp.sum(jnp.where(sel, g, 0.0), axis=1, keepdims=True)
        ams.append(am)
        wks.append(wk)
        cur = jnp.where(sel, NEG, cur)
    wsum = wks[0] + wks[1] + wks[2]
    col8 = lax.broadcasted_iota(jnp.int32, (TN, 8), 1)
    ti = jnp.where(col8 == 0, ams[0],
                   jnp.where(col8 == 1, ams[1],
                             jnp.where(col8 == 2, ams[2], 0)))
    tw = jnp.where(col8 == 0, wks[0] / wsum,
                   jnp.where(col8 == 1, wks[1] / wsum,
                             jnp.where(col8 == 2, wks[2] / wsum, 0.0)))
    ti_ref[...] = ti.astype(jnp.int32)
    tw_ref[...] = tw
    cnts = jnp.zeros((1, EL), jnp.int32)
    for k in range(K):
        oh = (lane == ams[k]).astype(jnp.int32)
        cnts = cnts + jnp.sum(oh, axis=0, keepdims=True)
    cntb = jnp.broadcast_to(cnts, (8, EL)).astype(jnp.int32)
    gn = g / jnp.sum(g, axis=1, keepdims=True)
    psb = jnp.broadcast_to(jnp.sum(gn, axis=0, keepdims=True), (8, EL))

    @pl.when(step == 0)
    def _():
        cnt_ref[...] = cntb
        ps_ref[...] = psb

    @pl.when(step != 0)
    def _():
        cnt_ref[...] = cnt_ref[...] + cntb
        ps_ref[...] = ps_ref[...] + psb


def _gating(xf, gate_w_p, gate_b_p, bias_p):
    nb = N // TN
    return pl.pallas_call(
        _gating_body,
        grid=(nb,),
        in_specs=[
            pl.BlockSpec((TN, D), lambda i: (i, 0)),
            pl.BlockSpec((D, EL), lambda i: (0, 0)),
            pl.BlockSpec((1, EL), lambda i: (0, 0)),
            pl.BlockSpec((1, EL), lambda i: (0, 0)),
        ],
        out_specs=[
            pl.BlockSpec((TN, 8), lambda i: (i, 0)),
            pl.BlockSpec((TN, 8), lambda i: (i, 0)),
            pl.BlockSpec((8, EL), lambda i: (0, 0)),
            pl.BlockSpec((8, EL), lambda i: (0, 0)),
        ],
        out_shape=[
            jax.ShapeDtypeStruct((N, 8), jnp.int32),
            jax.ShapeDtypeStruct((N, 8), jnp.float32),
            jax.ShapeDtypeStruct((8, EL), jnp.int32),
            jax.ShapeDtypeStruct((8, EL), jnp.float32),
        ],
    )(xf, gate_w_p, gate_b_p, bias_p)


def _routing_body(ti_ref, cnt_ref, ps_ref, slots_ref, be_ref, aux_ref,
                  carry_ref, offs_ref):
    step = pl.program_id(0)

    @pl.when(step == 0)
    def _():
        cnt_row = cnt_ref[0:1, :].astype(jnp.float32)
        padded = jnp.ceil(cnt_row / BLK) * BLK
        r0 = lax.broadcasted_iota(jnp.int32, (EL, EL), 0)
        r1 = lax.broadcasted_iota(jnp.int32, (EL, EL), 1)
        eye = (r0 == r1).astype(jnp.float32)
        padded_col = lax.dot_general(
            eye, padded, (((1,), (1,)), ((), ())),
            preferred_element_type=jnp.float32)
        pc32 = padded_col[:32, :]
        s0 = lax.broadcasted_iota(jnp.int32, (32, 32), 0)
        s1 = lax.broadcasted_iota(jnp.int32, (32, 32), 1)
        tri = (s1 < s0).astype(jnp.float32)
        offs_col = jnp.dot(tri, pc32,
                           preferred_element_type=jnp.float32)
        offs_ref[...] = jnp.broadcast_to(offs_col, (32, EL))
        carry_ref[...] = jnp.zeros((32, EL), jnp.float32)
        offs_e = jnp.broadcast_to(offs_col[:E + 9, :], (32, EL))
        gmul = lax.broadcasted_iota(jnp.int32, (32, EL), 1).astype(
            jnp.float32) * BLK
        sub = lax.broadcasted_iota(jnp.int32, (32, EL), 0)
        cmp = jnp.where((sub < E) & (offs_ref[...] <= gmul), 1.0, 0.0)
        be_row = jnp.sum(cmp, axis=0, keepdims=True) - 1.0
        be_row = jnp.clip(be_row, 0.0, float(E - 1))
        be_ref[...] = jnp.broadcast_to(be_row, (8, EL)).astype(jnp.int32)
        lane = lax.broadcasted_iota(jnp.int32, (1, EL), 1)
        p_mean = ps_ref[0:1, :] / N
        fv = (E * cnt_row) / (K * N)
        aux = jnp.sum(jnp.where(lane < E, p_mean * fv, 0.0))
        aux_ref[...] = jnp.full((8, EL), aux, jnp.float32)

    e_row = ti_ref[...].reshape(1, BLK)
    sub_e = lax.broadcasted_iota(jnp.int32, (32, BLK), 0)
    oh = (sub_e == e_row).astype(jnp.float32)
    p0 = lax.broadcasted_iota(jnp.int32, (BLK, BLK), 0)
    p1 = lax.broadcasted_iota(jnp.int32, (BLK, BLK), 1)
    tri = (p0 < p1).astype(jnp.float32)
    prefix = jnp.dot(oh, tri, preferred_element_type=jnp.float32)
    rank_in = jnp.sum(oh * prefix, axis=0, keepdims=True)
    base = jnp.sum(
        oh * (carry_ref[:, 0:1] + offs_ref[:, 0:1]), axis=0, keepdims=True)
    slot = (rank_in + base).astype(jnp.int32)
    slots_ref[...] = slot.reshape(1, 1, BLK)
    carry_new = carry_ref[:, 0:1] + jnp.sum(oh, axis=1, keepdims=True)
    carry_ref[...] = jnp.broadcast_to(carry_new, (32, EL))


def _routing(ti_k, cnt8, ps8):
    return pl.pallas_call(
        _routing_body,
        grid=(PB,),
        in_specs=[
            pl.BlockSpec((1, 1, BLK), lambda s: (s, 0, 0)),
            pl.BlockSpec((8, EL), lambda s: (0, 0)),
            pl.BlockSpec((8, EL), lambda s: (0, 0)),
        ],
        out_specs=[
            pl.BlockSpec((1, 1, BLK), lambda s: (s, 0, 0)),
            pl.BlockSpec((8, EL), lambda s: (0, 0)),
            pl.BlockSpec((8, EL), lambda s: (0, 0)),
        ],
        out_shape=[
            jax.ShapeDtypeStruct((PB, 1, BLK), jnp.int32),
            jax.ShapeDtypeStruct((8, EL), jnp.int32),
            jax.ShapeDtypeStruct((8, EL), jnp.float32),
        ],
        scratch_shapes=[
            pltpu.VMEM((32, EL), jnp.float32),
            pltpu.VMEM((32, EL), jnp.float32),
        ],
    )(ti_k, cnt8, ps8)


def _expert_body(be_ref, x_ref, w1_ref, b1_ref, w2_ref, b2_ref, y_ref):
    x = x_ref[...]
    h = jnp.dot(x, w1_ref[0], preferred_element_type=jnp.float32) + b1_ref[0]
    h = _gelu_exact(h)
    y_ref[...] = (jnp.dot(h, w2_ref[0], preferred_element_type=jnp.float32)
                  + b2_ref[0])


def _expert_mlp(be, padded_x, uw1, ub1r, uw2, ub2r):
    grid_spec = pltpu.PrefetchScalarGridSpec(
        num_scalar_prefetch=1,
        grid=(G,),
        in_specs=[
            pl.BlockSpec((BLK, D), lambda g, be: (g, 0)),
            pl.BlockSpec((1, D, H), lambda g, be: (be[g], 0, 0)),
            pl.BlockSpec((1, 1, H), lambda g, be: (be[g], 0, 0)),
            pl.BlockSpec((1, H, D), lambda g, be: (be[g], 0, 0)),
            pl.BlockSpec((1, 1, D), lambda g, be: (be[g], 0, 0)),
        ],
        out_specs=pl.BlockSpec((BLK, D), lambda g, be: (g, 0)),
    )
    return pl.pallas_call(
        _expert_body,
        grid_spec=grid_spec,
        out_shape=jax.ShapeDtypeStruct((NSLOT, D), jnp.float32),
    )(be, padded_x, uw1, ub1r, uw2, ub2r)


_SB = 512


def _final_body(x_ref, w1_ref, b1_ref, w2_ref, b2_ref, yg_ref, tw_ref, o_ref):
    x = x_ref[...]
    h = jnp.dot(x, w1_ref[...], preferred_element_type=jnp.float32) + b1_ref[...]
    h = _gelu_exact(h)
    s = (jnp.dot(h, w2_ref[...], preferred_element_type=jnp.float32)
         + b2_ref[...])
    tw = tw_ref[...]
    o_ref[...] = (s
                  + tw[:, 0:1] * yg_ref[0]
                  + tw[:, 1:2] * yg_ref[1]
                  + tw[:, 2:3] * yg_ref[2])


def _final(xf, sw1, sb1r, sw2, sb2r, yg, tw8):
    return pl.pallas_call(
        _final_body,
        grid=(N // _SB,),
        in_specs=[
            pl.BlockSpec((_SB, D), lambda i: (i, 0)),
            pl.BlockSpec((D, H), lambda i: (0, 0)),
            pl.BlockSpec((1, H), lambda i: (0, 0)),
            pl.BlockSpec((H, D), lambda i: (0, 0)),
            pl.BlockSpec((1, D), lambda i: (0, 0)),
            pl.BlockSpec((K, _SB, D), lambda i: (0, i, 0)),
            pl.BlockSpec((_SB, 8), lambda i: (i, 0)),
        ],
        out_specs=pl.BlockSpec((_SB, D), lambda i: (i, 0)),
        out_shape=jax.ShapeDtypeStruct((N, D), jnp.float32),
    )(xf, sw1, sb1r, sw2, sb2r, yg, tw8)


_NC = 2
_NS = 16
_NW = _NC * _NS
TPW = N // _NW
CT = 64


def _dispatch_body(x_hbm, slots_hbm, out_hbm, rows0, rows1, idx_all, sem0,
                   sem1):
    wid = lax.axis_index("s") * _NC + lax.axis_index("c")
    base = wid * TPW
    for k in range(K):
        pltpu.sync_copy(slots_hbm.at[pl.ds(k * N + base, TPW)],
                        idx_all.at[pl.ds(k * TPW, TPW)])
    bufs = (rows0, rows1)
    sems = (sem0, sem1)
    pend = [None, None]
    for c in range(TPW // CT):
        b = c % 2
        if pend[b] is not None:
            for h in pend[b]:
                h.wait()
        pltpu.sync_copy(x_hbm.at[pl.ds(base + c * CT, CT), :], bufs[b])
        hs = []
        for k in range(K):
            idx = idx_all.at[pl.ds(k * TPW + c * CT, CT)]
            hs.append(pltpu.async_copy(bufs[b], out_hbm.at[idx], sems[b]))
        pend[b] = hs
    for p in pend:
        if p is not None:
            for h in p:
                h.wait()


_dispatch = functools.partial(
    pl.kernel,
    mesh=plsc.VectorSubcoreMesh(core_axis_name="c", subcore_axis_name="s"),
    out_type=jax.ShapeDtypeStruct((NSLOT, D), jnp.float32),
    scratch_types=[
        pltpu.VMEM((CT, D), jnp.float32),
        pltpu.VMEM((CT, D), jnp.float32),
        pltpu.VMEM((K * TPW,), jnp.int32),
        pltpu.SemaphoreType.DMA,
        pltpu.SemaphoreType.DMA,
    ],
)(_dispatch_body)


def _combine_body(y_hbm, slots_hbm, out_hbm, idx_all, rows0, rows1, gsem0,
                  gsem1, wsem0, wsem1):
    wid = lax.axis_index("s") * _NC + lax.axis_index("c")
    base = wid * TPW
    for k in range(K):
        pltpu.sync_copy(slots_hbm.at[pl.ds(k * N + base, TPW)],
                        idx_all.at[pl.ds(k * TPW, TPW)])
    bufs = (rows0, rows1)
    gsems = (gsem0, gsem1)
    wsems = (wsem0, wsem1)
    gpend = [None, None]
    wpend = [None, None]
    steps = [(c, k) for c in range(TPW // CT) for k in range(K)]
    for i, (c, k) in enumerate(steps):
        b = i % 2
        if wpend[b] is not None:
            wpend[b].wait()
            wpend[b] = None
        idx = idx_all.at[pl.ds(k * TPW + c * CT, CT)]
        g = pltpu.async_copy(y_hbm.at[idx], bufs[b], gsems[b])
        o = 1 - b
        if gpend[o] is not None:
            h, pc, pk = gpend[o]
            h.wait()
            wpend[o] = pltpu.async_copy(
                bufs[o], out_hbm.at[pk, pl.ds(base + pc * CT, CT), :],
                wsems[o])
            gpend[o] = None
        gpend[b] = (g, c, k)
    for b in range(2):
        if gpend[b] is not None:
            h, pc, pk = gpend[b]
            h.wait()
            wpend[b] = pltpu.async_copy(
                bufs[b], out_hbm.at[pk, pl.ds(base + pc * CT, CT), :],
                wsems[b])
    for b in range(2):
        if wpend[b] is not None:
            wpend[b].wait()


_combine = functools.partial(
    pl.kernel,
    mesh=plsc.VectorSubcoreMesh(core_axis_name="c", subcore_axis_name="s"),
    out_type=jax.ShapeDtypeStruct((K, N, D), jnp.float32),
    scratch_types=[
        pltpu.VMEM((K * TPW,), jnp.int32),
        pltpu.VMEM((CT, D), jnp.float32),
        pltpu.VMEM((CT, D), jnp.float32),
        pltpu.SemaphoreType.DMA,
        pltpu.SemaphoreType.DMA,
        pltpu.SemaphoreType.DMA,
        pltpu.SemaphoreType.DMA,
    ],
)(_combine_body)


def kernel(x, gate_w, gate_b, bias_buf, uw1, ub1, uw2, ub2, sw1, sb1, sw2,
           sb2):
    o_shape = x.shape
    xf = x.reshape(N, D)

    gate_w_p = jnp.zeros((D, EL), jnp.float32).at[:, :E].set(gate_w)
    gate_b_p = jnp.zeros((1, EL), jnp.float32).at[:, :E].set(gate_b)
    bias_p = jnp.full((1, EL), NEG, jnp.float32).at[:, :E].set(bias_buf)

    ti8, tw8, cnt8, ps8 = _gating(xf, gate_w_p, gate_b_p, bias_p)

    ti_k = jnp.transpose(ti8[:, :K]).reshape(PB, 1, BLK)
    slots_b, be8, aux8 = _routing(ti_k, cnt8, ps8)
    slots = slots_b.reshape(K, N)
    be = be8[0, :G]

    slots_flat = slots.reshape(K * N)
    padded_x = _dispatch(xf, slots_flat)
    y = _expert_mlp(be, padded_x, uw1, ub1.reshape(E, 1, H), uw2,
                    ub2.reshape(E, 1, D))
    yg = _combine(y, slots_flat)
    out = _final(xf, sw1, sb1.reshape(1, H), sw2, sb2.reshape(1, D), yg, tw8)

    return (out.reshape(o_shape), aux8[0, 0], cnt8[0, :E])


# speedup vs baseline: 1.1377x; 1.0819x over previous
import functools

import jax
import jax.numpy as jnp
from jax import lax
from jax.experimental import pallas as pl
from jax.experimental.pallas import tpu as pltpu
from jax.experimental.pallas import tpu_sc as plsc

E = 23
K = 3
D = 768
H = 768
N = 8192
EL = 128
BLK = 256
G = 118
NSLOT = G * BLK
TN = 1024
RB = 2048
PB = (N * K) // RB

NEG = -1e30

_INV_SQRT2 = 0.7071067811865476


def _gelu_exact(v):
    return 0.5 * v * (1.0 + lax.erf(v * _INV_SQRT2))


def _gating_body(x_ref, gw_ref, gb_ref, bias_ref, ti_ref, tw_ref, cnt_ref,
                 ps_ref):
    step = pl.program_id(0)
    x = x_ref[...]
    logits = jnp.dot(x, gw_ref[...], preferred_element_type=jnp.float32)
    logits = logits + gb_ref[...]
    lane = lax.broadcasted_iota(jnp.int32, (TN, EL), 1)
    valid = lane < E
    g = jnp.where(valid, jax.nn.sigmoid(logits), 0.0)
    cur = jnp.where(valid, g + bias_ref[...], NEG)
    ams = []
    wks = []
    for _ in range(K):
        m = jnp.max(cur, axis=1, keepdims=True)
        am = jnp.min(jnp.where(cur == m, lane, EL), axis=1, keepdims=True)
        sel = lane == am
        wk = jnp.sum(jnp.where(sel, g, 0.0), axis=1, keepdims=True)
        ams.append(am)
        wks.append(wk)
        cur = jnp.where(sel, NEG, cur)
    wsum = wks[0] + wks[1] + wks[2]
    col8 = lax.broadcasted_iota(jnp.int32, (TN, 8), 1)
    ti = jnp.where(col8 == 0, ams[0],
                   jnp.where(col8 == 1, ams[1],
                             jnp.where(col8 == 2, ams[2], 0)))
    tw = jnp.where(col8 == 0, wks[0] / wsum,
                   jnp.where(col8 == 1, wks[1] / wsum,
                             jnp.where(col8 == 2, wks[2] / wsum, 0.0)))
    ti_ref[...] = ti.astype(jnp.int32)
    tw_ref[...] = tw
    cnts = jnp.zeros((1, EL), jnp.int32)
    for k in range(K):
        oh = (lane == ams[k]).astype(jnp.int32)
        cnts = cnts + jnp.sum(oh, axis=0, keepdims=True)
    cntb = jnp.broadcast_to(cnts, (8, EL)).astype(jnp.int32)
    gn = g / jnp.sum(g, axis=1, keepdims=True)
    psb = jnp.broadcast_to(jnp.sum(gn, axis=0, keepdims=True), (8, EL))

    @pl.when(step == 0)
    def _():
        cnt_ref[...] = cntb
        ps_ref[...] = psb

    @pl.when(step != 0)
    def _():
        cnt_ref[...] = cnt_ref[...] + cntb
        ps_ref[...] = ps_ref[...] + psb


def _gating(xf, gate_w_p, gate_b_p, bias_p):
    nb = N // TN
    return pl.pallas_call(
        _gating_body,
        grid=(nb,),
        in_specs=[
            pl.BlockSpec((TN, D), lambda i: (i, 0)),
            pl.BlockSpec((D, EL), lambda i: (0, 0)),
            pl.BlockSpec((1, EL), lambda i: (0, 0)),
            pl.BlockSpec((1, EL), lambda i: (0, 0)),
        ],
        out_specs=[
            pl.BlockSpec((TN, 8), lambda i: (i, 0)),
            pl.BlockSpec((TN, 8), lambda i: (i, 0)),
            pl.BlockSpec((8, EL), lambda i: (0, 0)),
            pl.BlockSpec((8, EL), lambda i: (0, 0)),
        ],
        out_shape=[
            jax.ShapeDtypeStruct((N, 8), jnp.int32),
            jax.ShapeDtypeStruct((N, 8), jnp.float32),
            jax.ShapeDtypeStruct((8, EL), jnp.int32),
            jax.ShapeDtypeStruct((8, EL), jnp.float32),
        ],
    )(xf, gate_w_p, gate_b_p, bias_p)


def _routing_body(ti_ref, cnt_ref, ps_ref, slots_ref, be_ref, aux_ref,
                  carry_ref, offs_ref):
    step = pl.program_id(0)

    @pl.when(step == 0)
    def _():
        cnt_row = cnt_ref[0:1, :].astype(jnp.float32)
        padded = jnp.ceil(cnt_row / BLK) * BLK
        r0 = lax.broadcasted_iota(jnp.int32, (EL, EL), 0)
        r1 = lax.broadcasted_iota(jnp.int32, (EL, EL), 1)
        eye = (r0 == r1).astype(jnp.float32)
        padded_col = lax.dot_general(
            eye, padded, (((1,), (1,)), ((), ())),
            preferred_element_type=jnp.float32)
        pc32 = padded_col[:32, :]
        s0 = lax.broadcasted_iota(jnp.int32, (32, 32), 0)
        s1 = lax.broadcasted_iota(jnp.int32, (32, 32), 1)
        tri = (s1 < s0).astype(jnp.float32)
        offs_col = jnp.dot(tri, pc32,
                           preferred_element_type=jnp.float32)
        offs_ref[...] = jnp.broadcast_to(offs_col, (32, EL))
        carry_ref[...] = jnp.zeros((32, EL), jnp.float32)
        offs_e = jnp.broadcast_to(offs_col[:E + 9, :], (32, EL))
        gmul = lax.broadcasted_iota(jnp.int32, (32, EL), 1).astype(
            jnp.float32) * BLK
        sub = lax.broadcasted_iota(jnp.int32, (32, EL), 0)
        cmp = jnp.where((sub < E) & (offs_ref[...] <= gmul), 1.0, 0.0)
        be_row = jnp.sum(cmp, axis=0, keepdims=True) - 1.0
        be_row = jnp.clip(be_row, 0.0, float(E - 1))
        be_ref[...] = jnp.broadcast_to(be_row, (8, EL)).astype(jnp.int32)
        lane = lax.broadcasted_iota(jnp.int32, (1, EL), 1)
        p_mean = ps_ref[0:1, :] / N
        fv = (E * cnt_row) / (K * N)
        aux = jnp.sum(jnp.where(lane < E, p_mean * fv, 0.0))
        aux_ref[...] = jnp.full((8, EL), aux, jnp.float32)

    e_row = ti_ref[...].reshape(1, RB)
    sub_e = lax.broadcasted_iota(jnp.int32, (32, RB), 0)
    oh = (sub_e == e_row).astype(jnp.float32)
    p0 = lax.broadcasted_iota(jnp.int32, (RB, RB), 0)
    p1 = lax.broadcasted_iota(jnp.int32, (RB, RB), 1)
    tri = (p0 < p1).astype(jnp.float32)
    prefix = jnp.dot(oh, tri, preferred_element_type=jnp.float32)
    rank_in = jnp.sum(oh * prefix, axis=0, keepdims=True)
    base = jnp.sum(
        oh * (carry_ref[:, 0:1] + offs_ref[:, 0:1]), axis=0, keepdims=True)
    slot = (rank_in + base).astype(jnp.int32)
    slots_ref[...] = slot.reshape(1, 1, RB)
    carry_new = carry_ref[:, 0:1] + jnp.sum(oh, axis=1, keepdims=True)
    carry_ref[...] = jnp.broadcast_to(carry_new, (32, EL))


def _routing(ti_k, cnt8, ps8):
    return pl.pallas_call(
        _routing_body,
        grid=(PB,),
        in_specs=[
            pl.BlockSpec((1, 1, RB), lambda s: (s, 0, 0)),
            pl.BlockSpec((8, EL), lambda s: (0, 0)),
            pl.BlockSpec((8, EL), lambda s: (0, 0)),
        ],
        out_specs=[
            pl.BlockSpec((1, 1, RB), lambda s: (s, 0, 0)),
            pl.BlockSpec((8, EL), lambda s: (0, 0)),
            pl.BlockSpec((8, EL), lambda s: (0, 0)),
        ],
        out_shape=[
            jax.ShapeDtypeStruct((PB, 1, RB), jnp.int32),
            jax.ShapeDtypeStruct((8, EL), jnp.int32),
            jax.ShapeDtypeStruct((8, EL), jnp.float32),
        ],
        scratch_shapes=[
            pltpu.VMEM((32, EL), jnp.float32),
            pltpu.VMEM((32, EL), jnp.float32),
        ],
    )(ti_k, cnt8, ps8)


def _expert_body(be_ref, x_ref, w1_ref, b1_ref, w2_ref, b2_ref, y_ref):
    x = x_ref[...]
    h = jnp.dot(x, w1_ref[0], preferred_element_type=jnp.float32) + b1_ref[0]
    h = _gelu_exact(h)
    y = (jnp.dot(h, w2_ref[0], preferred_element_type=jnp.float32)
         + b2_ref[0])
    y_ref[...] = y


def _expert_mlp(be, padded_x, uw1, ub1r, uw2, ub2r):
    grid_spec = pltpu.PrefetchScalarGridSpec(
        num_scalar_prefetch=1,
        grid=(G,),
        in_specs=[
            pl.BlockSpec((BLK, D), lambda g, be: (g, 0)),
            pl.BlockSpec((1, D, H), lambda g, be: (be[g], 0, 0)),
            pl.BlockSpec((1, 1, H), lambda g, be: (be[g], 0, 0)),
            pl.BlockSpec((1, H, D), lambda g, be: (be[g], 0, 0)),
            pl.BlockSpec((1, 1, D), lambda g, be: (be[g], 0, 0)),
        ],
        out_specs=pl.BlockSpec((BLK, D), lambda g, be: (g, 0)),
    )
    return pl.pallas_call(
        _expert_body,
        grid_spec=grid_spec,
        out_shape=jax.ShapeDtypeStruct((NSLOT, D), jnp.float32),
    )(be, padded_x, uw1, ub1r, uw2, ub2r)


_SB = 512


def _final_body(x_ref, w1_ref, b1_ref, w2_ref, b2_ref, yg_ref, tw_ref, o_ref):
    x = x_ref[...]
    h = jnp.dot(x, w1_ref[...], preferred_element_type=jnp.float32) + b1_ref[...]
    h = _gelu_exact(h)
    s = (jnp.dot(h, w2_ref[...], preferred_element_type=jnp.float32)
         + b2_ref[...])
    tw = tw_ref[...]
    o_ref[...] = (s
                  + tw[:, 0:1] * yg_ref[0].astype(jnp.float32)
                  + tw[:, 1:2] * yg_ref[1].astype(jnp.float32)
                  + tw[:, 2:3] * yg_ref[2].astype(jnp.float32))


def _final(xf, sw1, sb1r, sw2, sb2r, yg, tw8):
    return pl.pallas_call(
        _final_body,
        grid=(N // _SB,),
        in_specs=[
            pl.BlockSpec((_SB, D), lambda i: (i, 0)),
            pl.BlockSpec((D, H), lambda i: (0, 0)),
            pl.BlockSpec((1, H), lambda i: (0, 0)),
            pl.BlockSpec((H, D), lambda i: (0, 0)),
            pl.BlockSpec((1, D), lambda i: (0, 0)),
            pl.BlockSpec((K, _SB, D), lambda i: (0, i, 0)),
            pl.BlockSpec((_SB, 8), lambda i: (i, 0)),
        ],
        out_specs=pl.BlockSpec((_SB, D), lambda i: (i, 0)),
        out_shape=jax.ShapeDtypeStruct((N, D), jnp.float32),
    )(xf, sw1, sb1r, sw2, sb2r, yg, tw8)


_NC = 2
_NS = 16
_NW = _NC * _NS
TPW = N // _NW
CT = 64


def _dispatch_body(x_hbm, slots_hbm, out_hbm, rows0, rows1, idx_all, sem0,
                   sem1):
    wid = lax.axis_index("s") * _NC + lax.axis_index("c")
    base = wid * TPW
    for k in range(K):
        pltpu.sync_copy(slots_hbm.at[pl.ds(k * N + base, TPW)],
                        idx_all.at[pl.ds(k * TPW, TPW)])
    bufs = (rows0, rows1)
    sems = (sem0, sem1)
    pend = [None, None]
    for c in range(TPW // CT):
        b = c % 2
        if pend[b] is not None:
            for h in pend[b]:
                h.wait()
        pltpu.sync_copy(x_hbm.at[pl.ds(base + c * CT, CT), :], bufs[b])
        hs = []
        for k in range(K):
            idx = idx_all.at[pl.ds(k * TPW + c * CT, CT)]
            hs.append(pltpu.async_copy(bufs[b], out_hbm.at[idx], sems[b]))
        pend[b] = hs
    for p in pend:
        if p is not None:
            for h in p:
                h.wait()


_dispatch = functools.partial(
    pl.kernel,
    mesh=plsc.VectorSubcoreMesh(core_axis_name="c", subcore_axis_name="s"),
    out_type=jax.ShapeDtypeStruct((NSLOT, D), jnp.float32),
    scratch_types=[
        pltpu.VMEM((CT, D), jnp.float32),
        pltpu.VMEM((CT, D), jnp.float32),
        pltpu.VMEM((K * TPW,), jnp.int32),
        pltpu.SemaphoreType.DMA,
        pltpu.SemaphoreType.DMA,
    ],
)(_dispatch_body)


def _combine_body(y_hbm, slots_hbm, out_hbm, idx_all, rows0, rows1, gsem0,
                  gsem1, wsem0, wsem1):
    wid = lax.axis_index("s") * _NC + lax.axis_index("c")
    base = wid * TPW
    for k in range(K):
        pltpu.sync_copy(slots_hbm.at[pl.ds(k * N + base, TPW)],
                        idx_all.at[pl.ds(k * TPW, TPW)])
    bufs = (rows0, rows1)
    gsems = (gsem0, gsem1)
    wsems = (wsem0, wsem1)
    gpend = [None, None]
    wpend = [None, None]
    steps = [(c, k) for c in range(TPW // CT) for k in range(K)]
    for i, (c, k) in enumerate(steps):
        b = i % 2
        if wpend[b] is not None:
            wpend[b].wait()
            wpend[b] = None
        idx = idx_all.at[pl.ds(k * TPW + c * CT, CT)]
        g = pltpu.async_copy(y_hbm.at[idx], bufs[b], gsems[b])
        o = 1 - b
        if gpend[o] is not None:
            h, pc, pk = gpend[o]
            h.wait()
            wpend[o] = pltpu.async_copy(
                bufs[o], out_hbm.at[pk, pl.ds(base + pc * CT, CT), :],
                wsems[o])
            gpend[o] = None
        gpend[b] = (g, c, k)
    for b in range(2):
        if gpend[b] is not None:
            h, pc, pk = gpend[b]
            h.wait()
            wpend[b] = pltpu.async_copy(
                bufs[b], out_hbm.at[pk, pl.ds(base + pc * CT, CT), :],
                wsems[b])
    for b in range(2):
        if wpend[b] is not None:
            wpend[b].wait()


_combine = functools.partial(
    pl.kernel,
    mesh=plsc.VectorSubcoreMesh(core_axis_name="c", subcore_axis_name="s"),
    out_type=jax.ShapeDtypeStruct((K, N, D), jnp.float32),
    scratch_types=[
        pltpu.VMEM((K * TPW,), jnp.int32),
        pltpu.VMEM((CT, D), jnp.float32),
        pltpu.VMEM((CT, D), jnp.float32),
        pltpu.SemaphoreType.DMA,
        pltpu.SemaphoreType.DMA,
        pltpu.SemaphoreType.DMA,
        pltpu.SemaphoreType.DMA,
    ],
)(_combine_body)


def kernel(x, gate_w, gate_b, bias_buf, uw1, ub1, uw2, ub2, sw1, sb1, sw2,
           sb2):
    o_shape = x.shape
    xf = x.reshape(N, D)

    gate_w_p = jnp.zeros((D, EL), jnp.float32).at[:, :E].set(gate_w)
    gate_b_p = jnp.zeros((1, EL), jnp.float32).at[:, :E].set(gate_b)
    bias_p = jnp.full((1, EL), NEG, jnp.float32).at[:, :E].set(bias_buf)

    ti8, tw8, cnt8, ps8 = _gating(xf, gate_w_p, gate_b_p, bias_p)

    ti_k = jnp.transpose(ti8[:, :K]).reshape(PB, 1, RB)
    slots_b, be8, aux8 = _routing(ti_k, cnt8, ps8)
    slots = slots_b.reshape(K, N)
    be = be8[0, :G]

    slots_flat = slots.reshape(K * N)
    padded_x = _dispatch(xf, slots_flat)
    y = _expert_mlp(be, padded_x, uw1, ub1.reshape(E, 1, H), uw2,
                    ub2.reshape(E, 1, D))
    yg = _combine(y, slots_flat)
    out = _final(xf, sw1, sb1.reshape(1, H), sw2, sb2.reshape(1, D), yg, tw8)

    return (out.reshape(o_shape), aux8[0, 0], cnt8[0, :E])
